# baseline (speedup 1.0000x reference)
import functools

import jax
import jax.numpy as jnp
from jax import lax
from jax.experimental import pallas as pl
from jax.experimental.pallas import tpu as pltpu
from jax.experimental.pallas import tpu_sc as plsc

_N = 10000
_NP = 10240
_E = 160000
_EP = 172032
_NG = 64
_EB = 2048
_R = 512
_EBK = 32
_NBLK = _EP // _EBK
_RPT = _NP // 16
_BPT = _NBLK // 16

_f32 = jnp.float32


def _pre_body(eat, wem1, bem1, wem2, bem2, we1, be1, we2, be2, we3, be3,
              e1o, e2o, e3o):
    i = pl.program_id(0)
    a = jnp.maximum(jnp.dot(eat[...], wem1[...],
                            preferred_element_type=_f32) + bem1[...], 0.0)
    ea = jnp.dot(a, wem2[...], preferred_element_type=_f32) + bem2[...]
    rows = lax.broadcasted_iota(jnp.int32, (_EB, 1), 0) + i * _EB
    valid = rows < _E
    e1 = jnp.dot(ea, we1[...], preferred_element_type=_f32) + be1[...]
    e1o[...] = jnp.where(valid, e1, -1e9)
    e2 = jnp.dot(ea, we2[...], preferred_element_type=_f32) + be2[...]
    e2 = jnp.where(valid, e2, -1e9)
    for c in range(4):
        e2o[c] = e2[:, c * 128:(c + 1) * 128]
    e3 = jnp.dot(ea, we3[...], preferred_element_type=_f32) + be3[...]
    e3 = jnp.where(valid, e3, -1e9)
    for c in range(8):
        e3o[c] = e3[:, c * 128:(c + 1) * 128]


def _full_spec(arr):
    nd = arr.ndim
    return pl.BlockSpec(arr.shape, lambda i, _nd=nd: (0,) * _nd)


def _pre_kernel(eap, wem1, bem1, wem2, bem2, we1, be1, we2, be2, we3, be3):
    weights = (wem1, bem1, wem2, bem2, we1, be1, we2, be2, we3, be3)
    return pl.pallas_call(
        _pre_body,
        grid=(_EP // _EB,),
        in_specs=[pl.BlockSpec((_EB, 8), lambda i: (i, 0))] +
                 [_full_spec(w) for w in weights],
        out_specs=[pl.BlockSpec((_EB, 128), lambda i: (i, 0)),
                   pl.BlockSpec((4, _EB, 128), lambda i: (0, i, 0)),
                   pl.BlockSpec((8, _EB, 128), lambda i: (0, i, 0))],
        out_shape=[jax.ShapeDtypeStruct((_EP, 128), _f32),
                   jax.ShapeDtypeStruct((4, _EP, 128), _f32),
                   jax.ShapeDtypeStruct((8, _EP, 128), _f32)],
    )(eap, *weights)


def _k1_body(C, h, a0, wn, nb, t_ref, s1_ref, s2_ref):
    i = pl.program_id(0)
    acc = None
    for c in range(C):
        m = jnp.dot(h[c] + a0[c], wn[c],
                    preferred_element_type=_f32)
        acc = m if acc is None else acc + m
    t = jnp.maximum(acc + nb[...], 0.0)
    rows = lax.broadcasted_iota(jnp.int32, (_R, 1), 0) + i * _R
    t = jnp.where(rows < _N, t, 0.0)
    t_ref[...] = t
    ps = jnp.sum(t, axis=0, keepdims=True)
    psq = jnp.sum(t * t, axis=0, keepdims=True)

    @pl.when(i == 0)
    def _():
        s1_ref[...] = ps
        s2_ref[...] = psq

    @pl.when(i > 0)
    def _():
        s1_ref[...] = s1_ref[...] + ps
        s2_ref[...] = s2_ref[...] + psq


def _k1_kernel(h3d, a03d, wn, nb, dout):
    C, _, w = h3d.shape
    return pl.pallas_call(
        functools.partial(_k1_body, C),
        grid=(_NP // _R,),
        in_specs=[pl.BlockSpec((C, _R, w), lambda i: (0, i, 0))] * 2 +
                 [_full_spec(wn), _full_spec(nb)],
        out_specs=[pl.BlockSpec((_R, dout), lambda i: (i, 0)),
                   pl.BlockSpec((1, dout), lambda i: (0, 0)),
                   pl.BlockSpec((1, dout), lambda i: (0, 0))],
        out_shape=[jax.ShapeDtypeStruct((_NP, dout), _f32),
                   jax.ShapeDtypeStruct((1, dout), _f32),
                   jax.ShapeDtypeStruct((1, dout), _f32)],
    )(h3d, a03d, wn, nb)


def _k2_body(n_chunks, t, s1, s2, g, b, out):
    m = s1[...] / float(_N)
    v = s2[...] / float(_N) - m * m
    scale = lax.rsqrt(v + 1e-5) * g[...]
    y = (t[...] - m) * scale + b[...]
    if n_chunks == 0:
        out[...] = y
    else:
        for c in range(n_chunks):
            out[c] = y[:, c * 128:(c + 1) * 128]


def _k2_kernel(t, s1, s2, g, b, n_chunks):
    dout = t.shape[1]
    if n_chunks == 0:
        out_specs = pl.BlockSpec((_R, dout), lambda i: (i, 0))
        out_shape = jax.ShapeDtypeStruct((_NP, dout), _f32)
    else:
        out_specs = pl.BlockSpec((n_chunks, _R, 128), lambda i: (0, i, 0))
        out_shape = jax.ShapeDtypeStruct((n_chunks, _NP, 128), _f32)
    return pl.pallas_call(
        functools.partial(_k2_body, n_chunks),
        grid=(_NP // _R,),
        in_specs=[pl.BlockSpec((_R, dout), lambda i: (i, 0)),
                  _full_spec(s1), _full_spec(s2),
                  _full_spec(g), _full_spec(b)],
        out_specs=out_specs,
        out_shape=out_shape,
    )(t, s1, s2, g, b)


def _pool_body(h3, bt, wf1, bf1, wf2, bf2, wf3, bf3, out_ref, psum, pcnt):
    i = pl.program_id(0)

    @pl.when(i == 0)
    def _():
        psum[...] = jnp.zeros_like(psum)
        pcnt[...] = jnp.zeros_like(pcnt)

    gid = lax.broadcasted_iota(jnp.int32, (_NG, _R), 0)
    oh = jnp.where(gid == bt[0], 1.0, 0.0)
    psum[...] = psum[...] + jnp.dot(oh, h3[...], preferred_element_type=_f32)
    cnt = jnp.sum(oh, axis=1, keepdims=True)
    pcnt[...] = pcnt[...] + lax.broadcast_in_dim(cnt, (_NG, 128), (0, 1))

    @pl.when(i == _NP // _R - 1)
    def _():
        den = jnp.maximum(pcnt[:, 0:1], 1.0)
        pooled = psum[...] / den
        r = jnp.maximum(jnp.dot(pooled, wf1[...],
                                preferred_element_type=_f32) + bf1[...], 0.0)
        r = jnp.maximum(jnp.dot(r, wf2[...],
                                preferred_element_type=_f32) + bf2[...], 0.0)
        out_ref[...] = jnp.dot(r, wf3[...],
                               preferred_element_type=_f32) + bf3[...]


def _pool_kernel(h3, batchp, wf1, bf1, wf2, bf2, wf3, bf3):
    weights = (wf1, bf1, wf2, bf2, wf3, bf3)
    return pl.pallas_call(
        _pool_body,
        grid=(_NP // _R,),
        in_specs=[pl.BlockSpec((_R, 2048), lambda i: (i, 0)),
                  pl.BlockSpec((1, 1, _R), lambda i: (i, 0, 0))] +
                 [_full_spec(w) for w in weights],
        out_specs=pl.BlockSpec((_NG, 128), lambda i: (0, 0)),
        out_shape=jax.ShapeDtypeStruct((_NG, 128), _f32),
        scratch_shapes=[pltpu.VMEM((_NG, 2048), _f32),
                        pltpu.VMEM((_NG, 128), _f32)],
    )(h3, batchp, *weights)


def _make_sc(C, W):
    mesh = plsc.VectorSubcoreMesh(core_axis_name="c", subcore_axis_name="s",
                                  num_cores=2, num_subcores=16)
    out_type = jax.ShapeDtypeStruct((C * _NP, W), _f32)
    NB = 3
    WIN = 168
    IW = WIN * _EBK
    scratch = [pltpu.VMEM((IW,), jnp.int32),
               pltpu.VMEM((IW,), jnp.int32)] + \
              [pltpu.VMEM((_EBK, W), _f32)] * (3 * NB) + \
              [pltpu.VMEM_SHARED((_NP, W), _f32)] + \
              [pltpu.SemaphoreType.DMA] * (3 * NB)

    def body(table, e, srcb, dstb, zz, out0, *sc):
        srcv, dstv = sc[0], sc[1]
        gbuf = sc[2:2 + NB]
        ebuf = sc[2 + NB:2 + 2 * NB]
        sbuf = sc[2 + 2 * NB:2 + 3 * NB]
        acc = sc[2 + 3 * NB]
        gsem = sc[3 + 3 * NB:3 + 4 * NB]
        esem = sc[3 + 4 * NB:3 + 5 * NB]
        ssem = sc[3 + 5 * NB:3 + 6 * NB]

        cid = lax.axis_index("c")
        sid = lax.axis_index("s")
        r0 = pl.multiple_of(sid * _RPT, 8)

        def process_window(blk0, n, coff, ceoff):
            i0 = pl.multiple_of(blk0 * _EBK, 128)
            pltpu.sync_copy(srcb.at[pl.ds(i0, n * _EBK)],
                            srcv.at[pl.ds(0, n * _EBK)])
            pltpu.sync_copy(dstb.at[pl.ds(i0, n * _EBK)],
                            dstv.at[pl.ds(0, n * _EBK)])

            def adj(q, carry):
                sl = pl.ds(pl.multiple_of(q * 16, 16), 16)
                srcv[sl] = srcv[sl] + coff
                return carry

            lax.fori_loop(0, n * _EBK // 16, adj, 0)

            def gslice(jj):
                return srcv.at[pl.ds(pl.multiple_of(jj * _EBK, _EBK), _EBK)]

            def g_start(jj, b):
                pltpu.async_copy(table.at[gslice(jj)], gbuf[b], gsem[b])

            def g_wait(jj, b):
                pltpu.make_async_copy(table.at[gslice(jj)], gbuf[b],
                                      gsem[b]).wait()

            def eoff(jj):
                return pl.multiple_of(ceoff + (blk0 + jj) * _EBK, _EBK)

            def e_start(jj, b):
                pltpu.async_copy(e.at[pl.ds(eoff(jj), _EBK)], ebuf[b],
                                 esem[b])

            def e_wait(jj, b):
                pltpu.make_async_copy(e.at[pl.ds(eoff(jj), _EBK)], ebuf[b],
                                      esem[b]).wait()

            def s_start(jj, b):
                for k in range(_EBK // 16):
                    iv = dstv[pl.ds(jj * _EBK + k * 16, 16)]
                    pltpu.async_copy(sbuf[b].at[pl.ds(k * 16, 16)],
                                     acc.at[iv], ssem[b], add=True)

            def s_wait(jj, b):
                for k in range(_EBK // 16):
                    iv = dstv[pl.ds(jj * _EBK + k * 16, 16)]
                    pltpu.make_async_copy(sbuf[b].at[pl.ds(k * 16, 16)],
                                          acc.at[iv], ssem[b]).wait()

            for b in range(NB):
                g_start(b, b)
                e_start(b, b)

            def step(j, carry):
                for b in range(NB):
                    jj = NB * j + b
                    g_wait(jj, b)
                    e_wait(jj, b)

                    def rowf(r, cc):
                        for k in range(W // 16):
                            sl = pl.ds(k * 16, 16)
                            sbuf[b][r, sl] = jnp.maximum(
                                gbuf[b][r, sl] + ebuf[b][r, sl], 0.0)
                        return cc

                    lax.fori_loop(0, _EBK, rowf, 0)

                    @pl.when(jj >= NB)
                    def _():
                        s_wait(jj, b)

                    s_start(jj, b)

                    @pl.when(jj + NB < n)
                    def _():
                        g_start(jj + NB, b)
                        e_start(jj + NB, b)
                return carry

            lax.fori_loop(0, n // NB, step, 0)
            for b in range(NB):
                s_wait(n - NB + b, b)

        def chunk_body(c, carry):
            coff = c * _NP
            ceoff = c * _EP
            @pl.when(cid == 0)
            def _():
                pltpu.sync_copy(zz.at[pl.ds(r0, _RPT)],
                                acc.at[pl.ds(r0, _RPT)])

            plsc.subcore_barrier()

            @pl.when(cid == 0)
            def _():
                for win in range(_BPT // WIN):
                    process_window(sid * _BPT + win * WIN, WIN, coff, ceoff)

            plsc.subcore_barrier()
            o0 = pl.multiple_of(coff + r0, 8)

            @pl.when(cid == 0)
            def _():
                pltpu.sync_copy(acc.at[pl.ds(r0, _RPT)],
                                out0.at[pl.ds(o0, _RPT)])

            plsc.subcore_barrier()
            return carry

        lax.fori_loop(0, C, chunk_body, 0)

    return pl.kernel(body, out_type=out_type, mesh=mesh,
                     scratch_types=scratch)


def kernel(x, edge_index, edge_attr, batch, Wem1, bem1, Wem2, bem2, We1, be1,
           Wn1, nb1, g1, b1, We2, be2, Wn2, nb2, g2, b2, We3, be3, Wn3, nb3,
           g3, b3, Wf1, bf1, Wf2, bf2, Wf3, bf3):
    i32 = jnp.int32
    xp = jnp.zeros((_NP, 128), _f32).at[:_N, :6].set(x)
    eap = jnp.zeros((_EP, 8), _f32).at[:_E, :6].set(edge_attr)
    src = jnp.zeros((_EP,), i32).at[:_E].set(edge_index[0])
    dst = jnp.zeros((_EP,), i32).at[:_E].set(edge_index[1])
    batchp = jnp.full((_NP,), _NG, i32).at[:_N].set(batch).reshape(
        _NP // _R, 1, _R)
    z128 = jnp.zeros((_NP, 128), _f32)

    wem1p = jnp.zeros((8, 64), _f32).at[:6].set(Wem1)
    we1p = jnp.zeros((64, 128), _f32).at[:, :6].set(We1)
    be1p = jnp.zeros((1, 128), _f32).at[0, :6].set(be1)
    wn1p = jnp.zeros((1, 128, 512), _f32).at[0, :6].set(Wn1)
    wn2r = Wn2.reshape(4, 128, 1024)
    wn3r = Wn3.reshape(8, 128, 2048)
    wf3p = jnp.zeros((512, 128), _f32).at[:, :86].set(Wf3)
    bf3p = jnp.zeros((1, 128), _f32).at[0, :86].set(bf3)
    r1 = lambda a: a.reshape(1, -1)

    e1, e2c, e3c = _pre_kernel(eap, wem1p, r1(bem1), Wem2, r1(bem2),
                               we1p, be1p, We2, r1(be2), We3, r1(be3))

    a1 = _make_sc(1, 128)(xp, e1, src, dst, z128)
    t1, s11, s12 = _k1_kernel(xp.reshape(1, _NP, 128),
                              a1.reshape(1, _NP, 128), wn1p, r1(nb1), 512)
    h1c = _k2_kernel(t1, s11, s12, r1(g1), r1(b1), 4)

    a2 = _make_sc(4, 128)(h1c.reshape(4 * _NP, 128),
                          e2c.reshape(4 * _EP, 128), src, dst, z128)
    t2, s21, s22 = _k1_kernel(h1c, a2.reshape(4, _NP, 128), wn2r,
                              r1(nb2), 1024)
    h2c = _k2_kernel(t2, s21, s22, r1(g2), r1(b2), 8)

    a3 = _make_sc(8, 128)(h2c.reshape(8 * _NP, 128),
                          e3c.reshape(8 * _EP, 128), src, dst, z128)
    t3, s31, s32 = _k1_kernel(h2c, a3.reshape(8, _NP, 128), wn3r,
                              r1(nb3), 2048)
    h3 = _k2_kernel(t3, s31, s32, r1(g3), r1(b3), 0)

    outp = _pool_kernel(h3, batchp, Wf1, r1(bf1), Wf2, r1(bf2), wf3p, bf3p)
    return outp[:, :86]


# speedup vs baseline: 1.0002x; 1.0002x over previous
import functools

import jax
import jax.numpy as jnp
from jax import lax
from jax.experimental import pallas as pl
from jax.experimental.pallas import tpu as pltpu
from jax.experimental.pallas import tpu_sc as plsc

_N = 10000
_NP = 10240
_E = 160000
_EP = 172032
_NG = 64
_EB = 2048
_R = 512
_EBK = 32
_NBLK = _EP // _EBK
_RPT = _NP // 16
_BPT = _NBLK // 16

_f32 = jnp.float32


def _pre_body(eat, wem1, bem1, wem2, bem2, we1, be1, we2, be2, we3, be3,
              e1o, e2o, e3o):
    i = pl.program_id(0)
    a = jnp.maximum(jnp.dot(eat[...], wem1[...],
                            preferred_element_type=_f32) + bem1[...], 0.0)
    ea = jnp.dot(a, wem2[...], preferred_element_type=_f32) + bem2[...]
    rows = lax.broadcasted_iota(jnp.int32, (_EB, 1), 0) + i * _EB
    valid = rows < _E
    e1 = jnp.dot(ea, we1[...], preferred_element_type=_f32) + be1[...]
    e1o[...] = jnp.where(valid, e1, -1e9)
    e2 = jnp.dot(ea, we2[...], preferred_element_type=_f32) + be2[...]
    e2 = jnp.where(valid, e2, -1e9)
    for c in range(4):
        e2o[c] = e2[:, c * 128:(c + 1) * 128]
    e3 = jnp.dot(ea, we3[...], preferred_element_type=_f32) + be3[...]
    e3 = jnp.where(valid, e3, -1e9)
    for c in range(8):
        e3o[c] = e3[:, c * 128:(c + 1) * 128]


def _full_spec(arr):
    nd = arr.ndim
    return pl.BlockSpec(arr.shape, lambda i, _nd=nd: (0,) * _nd)


def _pre_kernel(eap, wem1, bem1, wem2, bem2, we1, be1, we2, be2, we3, be3):
    weights = (wem1, bem1, wem2, bem2, we1, be1, we2, be2, we3, be3)
    return pl.pallas_call(
        _pre_body,
        grid=(_EP // _EB,),
        in_specs=[pl.BlockSpec((_EB, 8), lambda i: (i, 0))] +
                 [_full_spec(w) for w in weights],
        out_specs=[pl.BlockSpec((_EB, 128), lambda i: (i, 0)),
                   pl.BlockSpec((4, _EB, 128), lambda i: (0, i, 0)),
                   pl.BlockSpec((8, _EB, 128), lambda i: (0, i, 0))],
        out_shape=[jax.ShapeDtypeStruct((_EP, 128), _f32),
                   jax.ShapeDtypeStruct((4, _EP, 128), _f32),
                   jax.ShapeDtypeStruct((8, _EP, 128), _f32)],
    )(eap, *weights)


def _k1_body(C, h, a0, wn, nb, t_ref, s1_ref, s2_ref):
    i = pl.program_id(0)
    acc = None
    for c in range(C):
        m = jnp.dot(h[c] + a0[c], wn[c],
                    preferred_element_type=_f32)
        acc = m if acc is None else acc + m
    t = jnp.maximum(acc + nb[...], 0.0)
    rows = lax.broadcasted_iota(jnp.int32, (_R, 1), 0) + i * _R
    t = jnp.where(rows < _N, t, 0.0)
    t_ref[...] = t
    ps = jnp.sum(t, axis=0, keepdims=True)
    psq = jnp.sum(t * t, axis=0, keepdims=True)

    @pl.when(i == 0)
    def _():
        s1_ref[...] = ps
        s2_ref[...] = psq

    @pl.when(i > 0)
    def _():
        s1_ref[...] = s1_ref[...] + ps
        s2_ref[...] = s2_ref[...] + psq


def _k1_kernel(h3d, a03d, wn, nb, dout):
    C, _, w = h3d.shape
    return pl.pallas_call(
        functools.partial(_k1_body, C),
        grid=(_NP // _R,),
        in_specs=[pl.BlockSpec((C, _R, w), lambda i: (0, i, 0))] * 2 +
                 [_full_spec(wn), _full_spec(nb)],
        out_specs=[pl.BlockSpec((_R, dout), lambda i: (i, 0)),
                   pl.BlockSpec((1, dout), lambda i: (0, 0)),
                   pl.BlockSpec((1, dout), lambda i: (0, 0))],
        out_shape=[jax.ShapeDtypeStruct((_NP, dout), _f32),
                   jax.ShapeDtypeStruct((1, dout), _f32),
                   jax.ShapeDtypeStruct((1, dout), _f32)],
    )(h3d, a03d, wn, nb)


def _k2_body(n_chunks, t, s1, s2, g, b, out):
    m = s1[...] / float(_N)
    v = s2[...] / float(_N) - m * m
    scale = lax.rsqrt(v + 1e-5) * g[...]
    y = (t[...] - m) * scale + b[...]
    if n_chunks == 0:
        out[...] = y
    else:
        for c in range(n_chunks):
            out[c] = y[:, c * 128:(c + 1) * 128]


def _k2_kernel(t, s1, s2, g, b, n_chunks):
    dout = t.shape[1]
    if n_chunks == 0:
        out_specs = pl.BlockSpec((_R, dout), lambda i: (i, 0))
        out_shape = jax.ShapeDtypeStruct((_NP, dout), _f32)
    else:
        out_specs = pl.BlockSpec((n_chunks, _R, 128), lambda i: (0, i, 0))
        out_shape = jax.ShapeDtypeStruct((n_chunks, _NP, 128), _f32)
    return pl.pallas_call(
        functools.partial(_k2_body, n_chunks),
        grid=(_NP // _R,),
        in_specs=[pl.BlockSpec((_R, dout), lambda i: (i, 0)),
                  _full_spec(s1), _full_spec(s2),
                  _full_spec(g), _full_spec(b)],
        out_specs=out_specs,
        out_shape=out_shape,
    )(t, s1, s2, g, b)


def _pool_body(h3, bt, wf1, bf1, wf2, bf2, wf3, bf3, out_ref, psum, pcnt):
    i = pl.program_id(0)

    @pl.when(i == 0)
    def _():
        psum[...] = jnp.zeros_like(psum)
        pcnt[...] = jnp.zeros_like(pcnt)

    gid = lax.broadcasted_iota(jnp.int32, (_NG, _R), 0)
    oh = jnp.where(gid == bt[0], 1.0, 0.0)
    psum[...] = psum[...] + jnp.dot(oh, h3[...], preferred_element_type=_f32)
    cnt = jnp.sum(oh, axis=1, keepdims=True)
    pcnt[...] = pcnt[...] + lax.broadcast_in_dim(cnt, (_NG, 128), (0, 1))

    @pl.when(i == _NP // _R - 1)
    def _():
        den = jnp.maximum(pcnt[:, 0:1], 1.0)
        pooled = psum[...] / den
        r = jnp.maximum(jnp.dot(pooled, wf1[...],
                                preferred_element_type=_f32) + bf1[...], 0.0)
        r = jnp.maximum(jnp.dot(r, wf2[...],
                                preferred_element_type=_f32) + bf2[...], 0.0)
        out_ref[...] = jnp.dot(r, wf3[...],
                               preferred_element_type=_f32) + bf3[...]


def _pool_kernel(h3, batchp, wf1, bf1, wf2, bf2, wf3, bf3):
    weights = (wf1, bf1, wf2, bf2, wf3, bf3)
    return pl.pallas_call(
        _pool_body,
        grid=(_NP // _R,),
        in_specs=[pl.BlockSpec((_R, 2048), lambda i: (i, 0)),
                  pl.BlockSpec((1, 1, _R), lambda i: (i, 0, 0))] +
                 [_full_spec(w) for w in weights],
        out_specs=pl.BlockSpec((_NG, 128), lambda i: (0, 0)),
        out_shape=jax.ShapeDtypeStruct((_NG, 128), _f32),
        scratch_shapes=[pltpu.VMEM((_NG, 2048), _f32),
                        pltpu.VMEM((_NG, 128), _f32)],
    )(h3, batchp, *weights)


def _make_sc(C, W):
    mesh = plsc.VectorSubcoreMesh(core_axis_name="c", subcore_axis_name="s",
                                  num_cores=2, num_subcores=16)
    out_type = jax.ShapeDtypeStruct((C * _NP, W), _f32)
    NB = 3
    WIN = 168
    IW = WIN * _EBK
    scratch = [pltpu.VMEM((IW,), jnp.int32),
               pltpu.VMEM((IW,), jnp.int32)] + \
              [pltpu.VMEM((_EBK, W), _f32)] * (3 * NB) + \
              [pltpu.VMEM_SHARED((_NP, W), _f32)] + \
              [pltpu.SemaphoreType.DMA] * (3 * NB)

    def body(table, e, srcb, dstb, zz, out0, *sc):
        srcv, dstv = sc[0], sc[1]
        gbuf = sc[2:2 + NB]
        ebuf = sc[2 + NB:2 + 2 * NB]
        sbuf = sc[2 + 2 * NB:2 + 3 * NB]
        acc = sc[2 + 3 * NB]
        gsem = sc[3 + 3 * NB:3 + 4 * NB]
        esem = sc[3 + 4 * NB:3 + 5 * NB]
        ssem = sc[3 + 5 * NB:3 + 6 * NB]

        cid = lax.axis_index("c")
        sid = lax.axis_index("s")
        r0 = pl.multiple_of(sid * _RPT, 8)

        def process_window(blk0, n, coff, ceoff):
            i0 = pl.multiple_of(blk0 * _EBK, 128)
            pltpu.sync_copy(srcb.at[pl.ds(i0, n * _EBK)],
                            srcv.at[pl.ds(0, n * _EBK)])
            pltpu.sync_copy(dstb.at[pl.ds(i0, n * _EBK)],
                            dstv.at[pl.ds(0, n * _EBK)])

            def adj(q, carry):
                sl = pl.ds(pl.multiple_of(q * 16, 16), 16)
                srcv[sl] = srcv[sl] + coff
                return carry

            lax.fori_loop(0, n * _EBK // 16, adj, 0)

            def gslice(jj):
                return srcv.at[pl.ds(pl.multiple_of(jj * _EBK, _EBK), _EBK)]

            def g_start(jj, b):
                pltpu.async_copy(table.at[gslice(jj)], gbuf[b], gsem[b])

            def g_wait(jj, b):
                pltpu.make_async_copy(table.at[gslice(jj)], gbuf[b],
                                      gsem[b]).wait()

            def eoff(jj):
                return pl.multiple_of(ceoff + (blk0 + jj) * _EBK, _EBK)

            def e_start(jj, b):
                pltpu.async_copy(e.at[pl.ds(eoff(jj), _EBK)], ebuf[b],
                                 esem[b])

            def e_wait(jj, b):
                pltpu.make_async_copy(e.at[pl.ds(eoff(jj), _EBK)], ebuf[b],
                                      esem[b]).wait()

            def s_start(jj, b):
                for k in range(_EBK // 16):
                    iv = dstv[pl.ds(jj * _EBK + k * 16, 16)]
                    pltpu.async_copy(sbuf[b].at[pl.ds(k * 16, 16)],
                                     acc.at[iv], ssem[b], add=True)

            def s_wait(jj, b):
                for k in range(_EBK // 16):
                    iv = dstv[pl.ds(jj * _EBK + k * 16, 16)]
                    pltpu.make_async_copy(sbuf[b].at[pl.ds(k * 16, 16)],
                                          acc.at[iv], ssem[b]).wait()

            for b in range(NB):
                g_start(b, b)
                e_start(b, b)

            def step(j, carry):
                for b in range(NB):
                    jj = NB * j + b
                    g_wait(jj, b)
                    e_wait(jj, b)

                    @pl.when(jj >= NB)
                    def _():
                        s_wait(jj, b)

                    def rowf(r, cc):
                        for k in range(W // 16):
                            sl = pl.ds(k * 16, 16)
                            sbuf[b][r, sl] = jnp.maximum(
                                gbuf[b][r, sl] + ebuf[b][r, sl], 0.0)
                        return cc

                    lax.fori_loop(0, _EBK, rowf, 0)

                    s_start(jj, b)

                    @pl.when(jj + NB < n)
                    def _():
                        g_start(jj + NB, b)
                        e_start(jj + NB, b)
                return carry

            lax.fori_loop(0, n // NB, step, 0)
            for b in range(NB):
                s_wait(n - NB + b, b)

        def chunk_body(c, carry):
            coff = c * _NP
            ceoff = c * _EP
            @pl.when(cid == 0)
            def _():
                pltpu.sync_copy(zz.at[pl.ds(r0, _RPT)],
                                acc.at[pl.ds(r0, _RPT)])

            plsc.subcore_barrier()

            @pl.when(cid == 0)
            def _():
                for win in range(_BPT // WIN):
                    process_window(sid * _BPT + win * WIN, WIN, coff, ceoff)

            plsc.subcore_barrier()
            o0 = pl.multiple_of(coff + r0, 8)

            @pl.when(cid == 0)
            def _():
                pltpu.sync_copy(acc.at[pl.ds(r0, _RPT)],
                                out0.at[pl.ds(o0, _RPT)])

            plsc.subcore_barrier()
            return carry

        lax.fori_loop(0, C, chunk_body, 0)

    return pl.kernel(body, out_type=out_type, mesh=mesh,
                     scratch_types=scratch)


def kernel(x, edge_index, edge_attr, batch, Wem1, bem1, Wem2, bem2, We1, be1,
           Wn1, nb1, g1, b1, We2, be2, Wn2, nb2, g2, b2, We3, be3, Wn3, nb3,
           g3, b3, Wf1, bf1, Wf2, bf2, Wf3, bf3):
    i32 = jnp.int32
    xp = jnp.zeros((_NP, 128), _f32).at[:_N, :6].set(x)
    eap = jnp.zeros((_EP, 8), _f32).at[:_E, :6].set(edge_attr)
    src = jnp.zeros((_EP,), i32).at[:_E].set(edge_index[0])
    dst = jnp.zeros((_EP,), i32).at[:_E].set(edge_index[1])
    batchp = jnp.full((_NP,), _NG, i32).at[:_N].set(batch).reshape(
        _NP // _R, 1, _R)
    z128 = jnp.zeros((_NP, 128), _f32)

    wem1p = jnp.zeros((8, 64), _f32).at[:6].set(Wem1)
    we1p = jnp.zeros((64, 128), _f32).at[:, :6].set(We1)
    be1p = jnp.zeros((1, 128), _f32).at[0, :6].set(be1)
    wn1p = jnp.zeros((1, 128, 512), _f32).at[0, :6].set(Wn1)
    wn2r = Wn2.reshape(4, 128, 1024)
    wn3r = Wn3.reshape(8, 128, 2048)
    wf3p = jnp.zeros((512, 128), _f32).at[:, :86].set(Wf3)
    bf3p = jnp.zeros((1, 128), _f32).at[0, :86].set(bf3)
    r1 = lambda a: a.reshape(1, -1)

    e1, e2c, e3c = _pre_kernel(eap, wem1p, r1(bem1), Wem2, r1(bem2),
                               we1p, be1p, We2, r1(be2), We3, r1(be3))

    a1 = _make_sc(1, 128)(xp, e1, src, dst, z128)
    t1, s11, s12 = _k1_kernel(xp.reshape(1, _NP, 128),
                              a1.reshape(1, _NP, 128), wn1p, r1(nb1), 512)
    h1c = _k2_kernel(t1, s11, s12, r1(g1), r1(b1), 4)

    a2 = _make_sc(4, 128)(h1c.reshape(4 * _NP, 128),
                          e2c.reshape(4 * _EP, 128), src, dst, z128)
    t2, s21, s22 = _k1_kernel(h1c, a2.reshape(4, _NP, 128), wn2r,
                              r1(nb2), 1024)
    h2c = _k2_kernel(t2, s21, s22, r1(g2), r1(b2), 8)

    a3 = _make_sc(8, 128)(h2c.reshape(8 * _NP, 128),
                          e3c.reshape(8 * _EP, 128), src, dst, z128)
    t3, s31, s32 = _k1_kernel(h2c, a3.reshape(8, _NP, 128), wn3r,
                              r1(nb3), 2048)
    h3 = _k2_kernel(t3, s31, s32, r1(g3), r1(b3), 0)

    outp = _pool_kernel(h3, batchp, Wf1, r1(bf1), Wf2, r1(bf2), wf3p, bf3p)
    return outp[:, :86]


# speedup vs baseline: 1.0568x; 1.0566x over previous
import functools

import jax
import jax.numpy as jnp
from jax import lax
from jax.experimental import pallas as pl
from jax.experimental.pallas import tpu as pltpu
from jax.experimental.pallas import tpu_sc as plsc

_N = 10000
_NP = 10240
_E = 160000
_EP = 172032
_NG = 64
_EB = 2048
_R = 512
_EBK = 128
_NBLK = _EP // _EBK
_RPT = _NP // 16
_BPT = _NBLK // 32

_f32 = jnp.float32


def _pre_body(eat, wem1, bem1, wem2, bem2, we1, be1, we2, be2, we3, be3,
              e1o, e2o, e3o):
    i = pl.program_id(0)
    a = jnp.maximum(jnp.dot(eat[...], wem1[...],
                            preferred_element_type=_f32) + bem1[...], 0.0)
    ea = jnp.dot(a, wem2[...], preferred_element_type=_f32) + bem2[...]
    rows = lax.broadcasted_iota(jnp.int32, (_EB, 1), 0) + i * _EB
    valid = rows < _E
    e1 = jnp.dot(ea, we1[...], preferred_element_type=_f32) + be1[...]
    e1o[...] = jnp.where(valid, e1, -1e9)
    e2 = jnp.dot(ea, we2[...], preferred_element_type=_f32) + be2[...]
    e2 = jnp.where(valid, e2, -1e9)
    for c in range(4):
        e2o[c] = e2[:, c * 128:(c + 1) * 128]
    e3 = jnp.dot(ea, we3[...], preferred_element_type=_f32) + be3[...]
    e3 = jnp.where(valid, e3, -1e9)
    for c in range(8):
        e3o[c] = e3[:, c * 128:(c + 1) * 128]


def _full_spec(arr):
    nd = arr.ndim
    return pl.BlockSpec(arr.shape, lambda i, _nd=nd: (0,) * _nd)


def _pre_kernel(eap, wem1, bem1, wem2, bem2, we1, be1, we2, be2, we3, be3):
    weights = (wem1, bem1, wem2, bem2, we1, be1, we2, be2, we3, be3)
    return pl.pallas_call(
        _pre_body,
        grid=(_EP // _EB,),
        in_specs=[pl.BlockSpec((_EB, 8), lambda i: (i, 0))] +
                 [_full_spec(w) for w in weights],
        out_specs=[pl.BlockSpec((_EB, 128), lambda i: (i, 0)),
                   pl.BlockSpec((4, _EB, 128), lambda i: (0, i, 0)),
                   pl.BlockSpec((8, _EB, 128), lambda i: (0, i, 0))],
        out_shape=[jax.ShapeDtypeStruct((_EP, 128), _f32),
                   jax.ShapeDtypeStruct((4, _EP, 128), _f32),
                   jax.ShapeDtypeStruct((8, _EP, 128), _f32)],
    )(eap, *weights)


def _k1_body(C, h, a0, wn, nb, t_ref, s1_ref, s2_ref):
    i = pl.program_id(0)
    acc = None
    for c in range(C):
        m = jnp.dot(h[c] + a0[c], wn[c],
                    preferred_element_type=_f32)
        acc = m if acc is None else acc + m
    t = jnp.maximum(acc + nb[...], 0.0)
    rows = lax.broadcasted_iota(jnp.int32, (_R, 1), 0) + i * _R
    t = jnp.where(rows < _N, t, 0.0)
    t_ref[...] = t
    ps = jnp.sum(t, axis=0, keepdims=True)
    psq = jnp.sum(t * t, axis=0, keepdims=True)

    @pl.when(i == 0)
    def _():
        s1_ref[...] = ps
        s2_ref[...] = psq

    @pl.when(i > 0)
    def _():
        s1_ref[...] = s1_ref[...] + ps
        s2_ref[...] = s2_ref[...] + psq


def _k1_kernel(h3d, a03d, wn, nb, dout):
    C, _, w = h3d.shape
    return pl.pallas_call(
        functools.partial(_k1_body, C),
        grid=(_NP // _R,),
        in_specs=[pl.BlockSpec((C, _R, w), lambda i: (0, i, 0))] * 2 +
                 [_full_spec(wn), _full_spec(nb)],
        out_specs=[pl.BlockSpec((_R, dout), lambda i: (i, 0)),
                   pl.BlockSpec((1, dout), lambda i: (0, 0)),
                   pl.BlockSpec((1, dout), lambda i: (0, 0))],
        out_shape=[jax.ShapeDtypeStruct((_NP, dout), _f32),
                   jax.ShapeDtypeStruct((1, dout), _f32),
                   jax.ShapeDtypeStruct((1, dout), _f32)],
    )(h3d, a03d, wn, nb)


def _k2_body(n_chunks, t, s1, s2, g, b, out):
    m = s1[...] / float(_N)
    v = s2[...] / float(_N) - m * m
    scale = lax.rsqrt(v + 1e-5) * g[...]
    y = (t[...] - m) * scale + b[...]
    if n_chunks == 0:
        out[...] = y
    else:
        for c in range(n_chunks):
            out[c] = y[:, c * 128:(c + 1) * 128]


def _k2_kernel(t, s1, s2, g, b, n_chunks):
    dout = t.shape[1]
    if n_chunks == 0:
        out_specs = pl.BlockSpec((_R, dout), lambda i: (i, 0))
        out_shape = jax.ShapeDtypeStruct((_NP, dout), _f32)
    else:
        out_specs = pl.BlockSpec((n_chunks, _R, 128), lambda i: (0, i, 0))
        out_shape = jax.ShapeDtypeStruct((n_chunks, _NP, 128), _f32)
    return pl.pallas_call(
        functools.partial(_k2_body, n_chunks),
        grid=(_NP // _R,),
        in_specs=[pl.BlockSpec((_R, dout), lambda i: (i, 0)),
                  _full_spec(s1), _full_spec(s2),
                  _full_spec(g), _full_spec(b)],
        out_specs=out_specs,
        out_shape=out_shape,
    )(t, s1, s2, g, b)


def _pool_body(h3, bt, wf1, bf1, wf2, bf2, wf3, bf3, out_ref, psum, pcnt):
    i = pl.program_id(0)

    @pl.when(i == 0)
    def _():
        psum[...] = jnp.zeros_like(psum)
        pcnt[...] = jnp.zeros_like(pcnt)

    gid = lax.broadcasted_iota(jnp.int32, (_NG, _R), 0)
    oh = jnp.where(gid == bt[0], 1.0, 0.0)
    psum[...] = psum[...] + jnp.dot(oh, h3[...], preferred_element_type=_f32)
    cnt = jnp.sum(oh, axis=1, keepdims=True)
    pcnt[...] = pcnt[...] + lax.broadcast_in_dim(cnt, (_NG, 128), (0, 1))

    @pl.when(i == _NP // _R - 1)
    def _():
        den = jnp.maximum(pcnt[:, 0:1], 1.0)
        pooled = psum[...] / den
        r = jnp.maximum(jnp.dot(pooled, wf1[...],
                                preferred_element_type=_f32) + bf1[...], 0.0)
        r = jnp.maximum(jnp.dot(r, wf2[...],
                                preferred_element_type=_f32) + bf2[...], 0.0)
        out_ref[...] = jnp.dot(r, wf3[...],
                               preferred_element_type=_f32) + bf3[...]


def _pool_kernel(h3, batchp, wf1, bf1, wf2, bf2, wf3, bf3):
    weights = (wf1, bf1, wf2, bf2, wf3, bf3)
    return pl.pallas_call(
        _pool_body,
        grid=(_NP // _R,),
        in_specs=[pl.BlockSpec((_R, 2048), lambda i: (i, 0)),
                  pl.BlockSpec((1, 1, _R), lambda i: (i, 0, 0))] +
                 [_full_spec(w) for w in weights],
        out_specs=pl.BlockSpec((_NG, 128), lambda i: (0, 0)),
        out_shape=jax.ShapeDtypeStruct((_NG, 128), _f32),
        scratch_shapes=[pltpu.VMEM((_NG, 2048), _f32),
                        pltpu.VMEM((_NG, 128), _f32)],
    )(h3, batchp, *weights)


def _make_sc(C, W):
    mesh = plsc.VectorSubcoreMesh(core_axis_name="c", subcore_axis_name="s",
                                  num_cores=2, num_subcores=16)
    out_type = [jax.ShapeDtypeStruct((C * _NP, W), _f32) for _ in range(2)]
    IPT = _BPT * _EBK
    scratch = [pltpu.VMEM((IPT,), jnp.int32),
               pltpu.VMEM((IPT,), jnp.int32),
               pltpu.VMEM((_EBK, W), _f32),
               pltpu.VMEM((_EBK, W), _f32),
               pltpu.VMEM_SHARED((_NP, W), _f32),
               pltpu.SemaphoreType.DMA,
               pltpu.SemaphoreType.DMA]

    def body(table, e, srcb, dstb, zz, out0, out1, *sc):
        srcv, dstv, gbuf, ebuf, acc, gsem, esem = sc

        cid = lax.axis_index("c")
        sid = lax.axis_index("s")
        wid = cid * 16 + sid
        r0 = pl.multiple_of(sid * _RPT, 8)
        ibase = pl.multiple_of(wid * IPT, 128)
        blk0 = wid * _BPT

        def chunk_body(c, carry):
            coff = c * _NP
            ceoff = c * _EP
            pltpu.sync_copy(srcb.at[pl.ds(ibase, IPT)], srcv)
            pltpu.sync_copy(dstb.at[pl.ds(ibase, IPT)], dstv)

            def adj(q, cc):
                sl = pl.ds(pl.multiple_of(q * 16, 16), 16)
                srcv[sl] = srcv[sl] + coff
                return cc

            lax.fori_loop(0, IPT // 16, adj, 0)
            pltpu.sync_copy(zz.at[pl.ds(r0, _RPT)], acc.at[pl.ds(r0, _RPT)])
            plsc.subcore_barrier()

            def step(jj, cc):
                io = pl.multiple_of(jj * _EBK, _EBK)
                eo = pl.multiple_of(ceoff + (blk0 + jj) * _EBK, _EBK)
                gd = pltpu.async_copy(table.at[srcv.at[pl.ds(io, _EBK)]],
                                      gbuf, gsem)
                ed = pltpu.async_copy(e.at[pl.ds(eo, _EBK)], ebuf, esem)
                gd.wait()
                ed.wait()

                def rowf(r, c2):
                    for k in range(W // 16):
                        sl = pl.ds(k * 16, 16)
                        gbuf[r, sl] = jnp.maximum(gbuf[r, sl] + ebuf[r, sl],
                                                  0.0)
                    return c2

                lax.fori_loop(0, _EBK, rowf, 0)
                for k in range(_EBK // 16):
                    iv = dstv[pl.ds(jj * _EBK + k * 16, 16)]
                    pltpu.sync_copy(gbuf.at[pl.ds(k * 16, 16)],
                                    acc.at[iv], add=True)
                return cc

            lax.fori_loop(0, _BPT, step, 0)
            plsc.subcore_barrier()
            o0 = pl.multiple_of(coff + r0, 8)

            @pl.when(cid == 0)
            def _():
                pltpu.sync_copy(acc.at[pl.ds(r0, _RPT)],
                                out0.at[pl.ds(o0, _RPT)])

            @pl.when(cid == 1)
            def _():
                pltpu.sync_copy(acc.at[pl.ds(r0, _RPT)],
                                out1.at[pl.ds(o0, _RPT)])

            plsc.subcore_barrier()
            return carry

        lax.fori_loop(0, C, chunk_body, 0)

    return pl.kernel(body, out_type=out_type, mesh=mesh,
                     scratch_types=scratch)


def kernel(x, edge_index, edge_attr, batch, Wem1, bem1, Wem2, bem2, We1, be1,
           Wn1, nb1, g1, b1, We2, be2, Wn2, nb2, g2, b2, We3, be3, Wn3, nb3,
           g3, b3, Wf1, bf1, Wf2, bf2, Wf3, bf3):
    i32 = jnp.int32
    xp = jnp.zeros((_NP, 128), _f32).at[:_N, :6].set(x)
    eap = jnp.zeros((_EP, 8), _f32).at[:_E, :6].set(edge_attr)
    src = jnp.zeros((_EP,), i32).at[:_E].set(edge_index[0])
    dst = jnp.zeros((_EP,), i32).at[:_E].set(edge_index[1])
    batchp = jnp.full((_NP,), _NG, i32).at[:_N].set(batch).reshape(
        _NP // _R, 1, _R)
    z128 = jnp.zeros((_NP, 128), _f32)

    wem1p = jnp.zeros((8, 64), _f32).at[:6].set(Wem1)
    we1p = jnp.zeros((64, 128), _f32).at[:, :6].set(We1)
    be1p = jnp.zeros((1, 128), _f32).at[0, :6].set(be1)
    wn1p = jnp.zeros((1, 128, 512), _f32).at[0, :6].set(Wn1)
    wn2r = Wn2.reshape(4, 128, 1024)
    wn3r = Wn3.reshape(8, 128, 2048)
    wf3p = jnp.zeros((512, 128), _f32).at[:, :86].set(Wf3)
    bf3p = jnp.zeros((1, 128), _f32).at[0, :86].set(bf3)
    r1 = lambda a: a.reshape(1, -1)

    e1, e2c, e3c = _pre_kernel(eap, wem1p, r1(bem1), Wem2, r1(bem2),
                               we1p, be1p, We2, r1(be2), We3, r1(be3))

    a1 = _make_sc(1, 128)(xp, e1, src, dst, z128)
    a1s = a1[0] + a1[1]
    t1, s11, s12 = _k1_kernel(xp.reshape(1, _NP, 128),
                              a1s.reshape(1, _NP, 128), wn1p, r1(nb1), 512)
    h1c = _k2_kernel(t1, s11, s12, r1(g1), r1(b1), 4)

    a2 = _make_sc(4, 128)(h1c.reshape(4 * _NP, 128),
                          e2c.reshape(4 * _EP, 128), src, dst, z128)
    t2, s21, s22 = _k1_kernel(h1c, (a2[0] + a2[1]).reshape(4, _NP, 128),
                              wn2r, r1(nb2), 1024)
    h2c = _k2_kernel(t2, s21, s22, r1(g2), r1(b2), 8)

    a3 = _make_sc(8, 128)(h2c.reshape(8 * _NP, 128),
                          e3c.reshape(8 * _EP, 128), src, dst, z128)
    t3, s31, s32 = _k1_kernel(h2c, (a3[0] + a3[1]).reshape(8, _NP, 128),
                              wn3r, r1(nb3), 2048)
    h3 = _k2_kernel(t3, s31, s32, r1(g3), r1(b3), 0)

    outp = _pool_kernel(h3, batchp, Wf1, r1(bf1), Wf2, r1(bf2), wf3p, bf3p)
    return outp[:, :86]


# speedup vs baseline: 1.8843x; 1.7830x over previous
import functools

import jax
import jax.numpy as jnp
from jax import lax
from jax.experimental import pallas as pl
from jax.experimental.pallas import tpu as pltpu
from jax.experimental.pallas import tpu_sc as plsc

_N = 10000
_NP = 10240
_E = 160000
_EP = 163840
_NG = 64
_EB = 2048
_R = 512
_EBK = 128
_NBLK = _EP // _EBK
_RPT = _NP // 16
_BPT = _NBLK // 32

_f32 = jnp.float32


def _pre_body(eat, wem1, bem1, wem2, bem2, we1, be1, we2, be2, we3, be3,
              e1o, e2o, e3o):
    i = pl.program_id(0)
    a = jnp.maximum(jnp.dot(eat[...], wem1[...],
                            preferred_element_type=_f32) + bem1[...], 0.0)
    ea = jnp.dot(a, wem2[...], preferred_element_type=_f32) + bem2[...]
    rows = lax.broadcasted_iota(jnp.int32, (_EB, 1), 0) + i * _EB
    valid = rows < _E
    e1 = jnp.dot(ea, we1[...], preferred_element_type=_f32) + be1[...]
    e1o[...] = jnp.where(valid, e1, -1e9)
    e2 = jnp.dot(ea, we2[...], preferred_element_type=_f32) + be2[...]
    e2 = jnp.where(valid, e2, -1e9)
    for c in range(4):
        e2o[c] = e2[:, c * 128:(c + 1) * 128]
    e3 = jnp.dot(ea, we3[...], preferred_element_type=_f32) + be3[...]
    e3 = jnp.where(valid, e3, -1e9)
    for c in range(8):
        e3o[c] = e3[:, c * 128:(c + 1) * 128]


def _full_spec(arr):
    nd = arr.ndim
    return pl.BlockSpec(arr.shape, lambda i, _nd=nd: (0,) * _nd)


def _pre_kernel(eap, wem1, bem1, wem2, bem2, we1, be1, we2, be2, we3, be3):
    weights = (wem1, bem1, wem2, bem2, we1, be1, we2, be2, we3, be3)
    return pl.pallas_call(
        _pre_body,
        grid=(_EP // _EB,),
        in_specs=[pl.BlockSpec((_EB, 8), lambda i: (i, 0))] +
                 [_full_spec(w) for w in weights],
        out_specs=[pl.BlockSpec((_EB, 128), lambda i: (i, 0)),
                   pl.BlockSpec((4, _EB, 128), lambda i: (0, i, 0)),
                   pl.BlockSpec((8, _EB, 128), lambda i: (0, i, 0))],
        out_shape=[jax.ShapeDtypeStruct((_EP, 128), _f32),
                   jax.ShapeDtypeStruct((4, _EP, 128), _f32),
                   jax.ShapeDtypeStruct((8, _EP, 128), _f32)],
    )(eap, *weights)


def _k1_body(C, h, a0, wn, nb, t_ref, s1_ref, s2_ref):
    i = pl.program_id(0)
    acc = None
    for c in range(C):
        m = jnp.dot(h[c] + a0[c], wn[c],
                    preferred_element_type=_f32)
        acc = m if acc is None else acc + m
    t = jnp.maximum(acc + nb[...], 0.0)
    rows = lax.broadcasted_iota(jnp.int32, (_R, 1), 0) + i * _R
    t = jnp.where(rows < _N, t, 0.0)
    t_ref[...] = t
    ps = jnp.sum(t, axis=0, keepdims=True)
    psq = jnp.sum(t * t, axis=0, keepdims=True)

    @pl.when(i == 0)
    def _():
        s1_ref[...] = ps
        s2_ref[...] = psq

    @pl.when(i > 0)
    def _():
        s1_ref[...] = s1_ref[...] + ps
        s2_ref[...] = s2_ref[...] + psq


def _k1_kernel(h3d, a03d, wn, nb, dout):
    C, _, w = h3d.shape
    return pl.pallas_call(
        functools.partial(_k1_body, C),
        grid=(_NP // _R,),
        in_specs=[pl.BlockSpec((C, _R, w), lambda i: (0, i, 0))] * 2 +
                 [_full_spec(wn), _full_spec(nb)],
        out_specs=[pl.BlockSpec((_R, dout), lambda i: (i, 0)),
                   pl.BlockSpec((1, dout), lambda i: (0, 0)),
                   pl.BlockSpec((1, dout), lambda i: (0, 0))],
        out_shape=[jax.ShapeDtypeStruct((_NP, dout), _f32),
                   jax.ShapeDtypeStruct((1, dout), _f32),
                   jax.ShapeDtypeStruct((1, dout), _f32)],
    )(h3d, a03d, wn, nb)


def _k2_body(n_chunks, t, s1, s2, g, b, out):
    m = s1[...] / float(_N)
    v = s2[...] / float(_N) - m * m
    scale = lax.rsqrt(v + 1e-5) * g[...]
    y = (t[...] - m) * scale + b[...]
    if n_chunks == 0:
        out[...] = y
    else:
        for c in range(n_chunks):
            out[c] = y[:, c * 128:(c + 1) * 128]


def _k2_kernel(t, s1, s2, g, b, n_chunks):
    dout = t.shape[1]
    if n_chunks == 0:
        out_specs = pl.BlockSpec((_R, dout), lambda i: (i, 0))
        out_shape = jax.ShapeDtypeStruct((_NP, dout), _f32)
    else:
        out_specs = pl.BlockSpec((n_chunks, _R, 128), lambda i: (0, i, 0))
        out_shape = jax.ShapeDtypeStruct((n_chunks, _NP, 128), _f32)
    return pl.pallas_call(
        functools.partial(_k2_body, n_chunks),
        grid=(_NP // _R,),
        in_specs=[pl.BlockSpec((_R, dout), lambda i: (i, 0)),
                  _full_spec(s1), _full_spec(s2),
                  _full_spec(g), _full_spec(b)],
        out_specs=out_specs,
        out_shape=out_shape,
    )(t, s1, s2, g, b)


def _pool_body(h3, bt, wf1, bf1, wf2, bf2, wf3, bf3, out_ref, psum, pcnt):
    i = pl.program_id(0)

    @pl.when(i == 0)
    def _():
        psum[...] = jnp.zeros_like(psum)
        pcnt[...] = jnp.zeros_like(pcnt)

    gid = lax.broadcasted_iota(jnp.int32, (_NG, _R), 0)
    oh = jnp.where(gid == bt[0], 1.0, 0.0)
    psum[...] = psum[...] + jnp.dot(oh, h3[...], preferred_element_type=_f32)
    cnt = jnp.sum(oh, axis=1, keepdims=True)
    pcnt[...] = pcnt[...] + lax.broadcast_in_dim(cnt, (_NG, 128), (0, 1))

    @pl.when(i == _NP // _R - 1)
    def _():
        den = jnp.maximum(pcnt[:, 0:1], 1.0)
        pooled = psum[...] / den
        r = jnp.maximum(jnp.dot(pooled, wf1[...],
                                preferred_element_type=_f32) + bf1[...], 0.0)
        r = jnp.maximum(jnp.dot(r, wf2[...],
                                preferred_element_type=_f32) + bf2[...], 0.0)
        out_ref[...] = jnp.dot(r, wf3[...],
                               preferred_element_type=_f32) + bf3[...]


def _pool_kernel(h3, batchp, wf1, bf1, wf2, bf2, wf3, bf3):
    weights = (wf1, bf1, wf2, bf2, wf3, bf3)
    return pl.pallas_call(
        _pool_body,
        grid=(_NP // _R,),
        in_specs=[pl.BlockSpec((_R, 2048), lambda i: (i, 0)),
                  pl.BlockSpec((1, 1, _R), lambda i: (i, 0, 0))] +
                 [_full_spec(w) for w in weights],
        out_specs=pl.BlockSpec((_NG, 128), lambda i: (0, 0)),
        out_shape=jax.ShapeDtypeStruct((_NG, 128), _f32),
        scratch_shapes=[pltpu.VMEM((_NG, 2048), _f32),
                        pltpu.VMEM((_NG, 128), _f32)],
    )(h3, batchp, *weights)


def _make_sc(C, W):
    mesh = plsc.VectorSubcoreMesh(core_axis_name="c", subcore_axis_name="s",
                                  num_cores=2, num_subcores=16)
    out_type = [jax.ShapeDtypeStruct((C * _NP, W), _f32) for _ in range(2)]
    IPT = _BPT * _EBK
    scratch = [pltpu.VMEM((IPT,), jnp.int32),
               pltpu.VMEM((_BPT, _EBK), jnp.int32),
               pltpu.VMEM((_EBK, W), _f32),
               pltpu.VMEM((_EBK, W), _f32),
               pltpu.VMEM_SHARED((_NP, W), _f32),
               pltpu.SemaphoreType.DMA,
               pltpu.SemaphoreType.DMA]

    def body(table, e, srcb, dstb, zz, out0, out1, *sc):
        srcv, dstv, gbuf, ebuf, acc, gsem, esem = sc

        cid = lax.axis_index("c")
        sid = lax.axis_index("s")
        wid = cid * 16 + sid
        r0 = pl.multiple_of(sid * _RPT, 8)
        ibase = pl.multiple_of(wid * IPT, 128)
        blk0 = wid * _BPT

        def chunk_body(c, carry):
            coff = c * _NP
            ceoff = c * _EP
            pltpu.sync_copy(srcb.at[pl.ds(ibase, IPT)], srcv)
            pltpu.sync_copy(dstb.at[pl.ds(blk0, _BPT)], dstv)

            def adj(q, cc):
                sl = pl.ds(pl.multiple_of(q * 16, 16), 16)
                srcv[sl] = srcv[sl] + coff
                return cc

            lax.fori_loop(0, IPT // 16, adj, 0)
            pltpu.sync_copy(zz.at[pl.ds(r0, _RPT)], acc.at[pl.ds(r0, _RPT)])
            plsc.subcore_barrier()

            def step(jj, cc):
                io = pl.multiple_of(jj * _EBK, _EBK)
                eo = pl.multiple_of(ceoff + (blk0 + jj) * _EBK, _EBK)
                gd = pltpu.async_copy(table.at[srcv.at[pl.ds(io, _EBK)]],
                                      gbuf, gsem)
                ed = pltpu.async_copy(e.at[pl.ds(eo, _EBK)], ebuf, esem)
                gd.wait()
                ed.wait()

                def rowf(r, c2):
                    for k in range(W // 16):
                        sl = pl.ds(k * 16, 16)
                        gbuf[r, sl] = jnp.maximum(gbuf[r, sl] + ebuf[r, sl],
                                                  0.0)
                    return c2

                lax.fori_loop(0, _EBK, rowf, 0)
                pltpu.sync_copy(gbuf, acc.at[dstv.at[jj]], add=True)
                return cc

            lax.fori_loop(0, _BPT, step, 0)
            plsc.subcore_barrier()
            o0 = pl.multiple_of(coff + r0, 8)

            @pl.when(cid == 0)
            def _():
                pltpu.sync_copy(acc.at[pl.ds(r0, _RPT)],
                                out0.at[pl.ds(o0, _RPT)])

            @pl.when(cid == 1)
            def _():
                pltpu.sync_copy(acc.at[pl.ds(r0, _RPT)],
                                out1.at[pl.ds(o0, _RPT)])

            plsc.subcore_barrier()
            return carry

        lax.fori_loop(0, C, chunk_body, 0)

    return pl.kernel(body, out_type=out_type, mesh=mesh,
                     scratch_types=scratch)


def kernel(x, edge_index, edge_attr, batch, Wem1, bem1, Wem2, bem2, We1, be1,
           Wn1, nb1, g1, b1, We2, be2, Wn2, nb2, g2, b2, We3, be3, Wn3, nb3,
           g3, b3, Wf1, bf1, Wf2, bf2, Wf3, bf3):
    i32 = jnp.int32
    xp = jnp.zeros((_NP, 128), _f32).at[:_N, :6].set(x)
    eap = jnp.zeros((_EP, 8), _f32).at[:_E, :6].set(edge_attr)
    src = jnp.zeros((_EP,), i32).at[:_E].set(edge_index[0])
    dst = jnp.zeros((_EP,), i32).at[:_E].set(edge_index[1]).reshape(_NBLK, _EBK)
    batchp = jnp.full((_NP,), _NG, i32).at[:_N].set(batch).reshape(
        _NP // _R, 1, _R)
    z128 = jnp.zeros((_NP, 128), _f32)

    wem1p = jnp.zeros((8, 64), _f32).at[:6].set(Wem1)
    we1p = jnp.zeros((64, 128), _f32).at[:, :6].set(We1)
    be1p = jnp.zeros((1, 128), _f32).at[0, :6].set(be1)
    wn1p = jnp.zeros((1, 128, 512), _f32).at[0, :6].set(Wn1)
    wn2r = Wn2.reshape(4, 128, 1024)
    wn3r = Wn3.reshape(8, 128, 2048)
    wf3p = jnp.zeros((512, 128), _f32).at[:, :86].set(Wf3)
    bf3p = jnp.zeros((1, 128), _f32).at[0, :86].set(bf3)
    r1 = lambda a: a.reshape(1, -1)

    e1, e2c, e3c = _pre_kernel(eap, wem1p, r1(bem1), Wem2, r1(bem2),
                               we1p, be1p, We2, r1(be2), We3, r1(be3))

    a1 = _make_sc(1, 128)(xp, e1, src, dst, z128)
    a1s = a1[0] + a1[1]
    t1, s11, s12 = _k1_kernel(xp.reshape(1, _NP, 128),
                              a1s.reshape(1, _NP, 128), wn1p, r1(nb1), 512)
    h1c = _k2_kernel(t1, s11, s12, r1(g1), r1(b1), 4)

    a2 = _make_sc(4, 128)(h1c.reshape(4 * _NP, 128),
                          e2c.reshape(4 * _EP, 128), src, dst, z128)
    t2, s21, s22 = _k1_kernel(h1c, (a2[0] + a2[1]).reshape(4, _NP, 128),
                              wn2r, r1(nb2), 1024)
    h2c = _k2_kernel(t2, s21, s22, r1(g2), r1(b2), 8)

    a3 = _make_sc(8, 128)(h2c.reshape(8 * _NP, 128),
                          e3c.reshape(8 * _EP, 128), src, dst, z128)
    t3, s31, s32 = _k1_kernel(h2c, (a3[0] + a3[1]).reshape(8, _NP, 128),
                              wn3r, r1(nb3), 2048)
    h3 = _k2_kernel(t3, s31, s32, r1(g3), r1(b3), 0)

    outp = _pool_kernel(h3, batchp, Wf1, r1(bf1), Wf2, r1(bf2), wf3p, bf3p)
    return outp[:, :86]
